# baseline (device time: 186695 ns/iter reference)
import jax
import jax.numpy as jnp
import numpy as np
from jax import lax
from jax.experimental import pallas as pl
from jax.experimental.pallas import tpu as pltpu

N_DEV = 8
SQ = 2048
D = 1024
HQ = 8
DH = 128
BLK = 512
CHUNK = SQ // N_DEV
HALF = D // 2
SCALE = 0.08838834764831843
N_HOP = N_DEV - 1


def _rope_tables():
    inv = 1.0 / (10000.0 ** (np.arange(0, DH, 2) / DH))
    pos = np.arange(SQ)[:, None] * inv[None, :]
    cos = np.repeat(np.cos(pos), 2, axis=-1).astype(np.float32)
    sin = np.repeat(np.sin(pos), 2, axis=-1).astype(np.float32)
    P = np.zeros((DH, DH), np.float32)
    for k in range(DH // 2):
        P[2 * k + 1, 2 * k] = -1.0
        P[2 * k, 2 * k + 1] = 1.0
    return cos, sin, P


_COS, _SIN, _P = _rope_tables()


def kernel(x, Wq, Wk, Wv, Wo):
    xb = x.reshape(SQ, D).astype(jnp.bfloat16)
    wq = Wq.astype(jnp.bfloat16)
    wk = Wk.astype(jnp.bfloat16)
    wv = Wv.astype(jnp.bfloat16)
    wo = Wo.astype(jnp.bfloat16)
    cos = jnp.asarray(_COS)
    sin = jnp.asarray(_SIN)
    pmat = jnp.asarray(_P, jnp.bfloat16)

    def body(x_ref, wq_ref, wk_ref, wv_ref, wo_ref, cos_ref, sin_ref, p_ref,
             out_ref, q_ref, k_ref, v_ref, ctx_ref, sbuf_p, comm_p,
             sbuf_m, comm_m, agstage_p, agcomm_p, agstage_m, agcomm_m,
             send_sems_p, recv_sems_p, send_sems_m, recv_sems_m):
        me = lax.axis_index("i")
        right = jnp.remainder(me + 1, N_DEV)

        xv = x_ref[...]
        q_ref[...] = jnp.dot(
            xv, wq_ref[...], preferred_element_type=jnp.float32
        ).astype(jnp.bfloat16)
        k_ref[...] = jnp.dot(
            xv, wk_ref[...], preferred_element_type=jnp.float32
        ).astype(jnp.bfloat16)
        v_ref[...] = jnp.dot(
            xv, wv_ref[...], preferred_element_type=jnp.float32
        ).astype(jnp.bfloat16)

        pm = p_ref[...]
        cos_f = cos_ref[...]
        sin_f = sin_ref[...]
        ones_kd = jnp.ones((SQ, DH), jnp.bfloat16)
        for h in range(HQ):
            c0 = h * DH
            kh = k_ref[:, c0:c0 + DH]
            krot = (
                kh.astype(jnp.float32) * cos_f
                + jnp.dot(kh, pm, preferred_element_type=jnp.float32) * sin_f
            ).astype(jnp.bfloat16)
            vh = v_ref[:, c0:c0 + DH]
            for rb in range(SQ // BLK):
                r0 = rb * BLK
                qh = q_ref[r0:r0 + BLK, c0:c0 + DH]
                qrot = (
                    (
                        qh.astype(jnp.float32) * cos_f[r0:r0 + BLK, :]
                        + jnp.dot(qh, pm, preferred_element_type=jnp.float32)
                        * sin_f[r0:r0 + BLK, :]
                    )
                    * SCALE
                ).astype(jnp.bfloat16)
                s = lax.dot_general(
                    qrot, krot, (((1,), (1,)), ((), ())),
                    preferred_element_type=jnp.float32,
                )
                e = jnp.exp(s).astype(jnp.bfloat16)
                denom = jnp.dot(
                    e, ones_kd, preferred_element_type=jnp.float32
                )
                pv = jnp.dot(e, vh, preferred_element_type=jnp.float32)
                ctx_ref[r0:r0 + BLK, c0:c0 + DH] = (pv / denom).astype(
                    jnp.bfloat16
                )

        out_ref[...] = jnp.dot(
            ctx_ref[...], wo_ref[...], preferred_element_type=jnp.float32
        )

        left = jnp.remainder(me - 1, N_DEV)

        for hop in range(N_HOP):
            p_send = pl.ds(jnp.remainder(me - hop, N_DEV) * CHUNK, CHUNK)
            m_send = pl.ds(jnp.remainder(me + hop, N_DEV) * CHUNK, CHUNK)
            sbuf_p[hop] = out_ref[p_send, 0:HALF].astype(jnp.bfloat16)
            sbuf_m[hop] = out_ref[m_send, HALF:D].astype(jnp.bfloat16)
            rdma_p = pltpu.make_async_remote_copy(
                src_ref=sbuf_p.at[hop],
                dst_ref=comm_p.at[hop],
                send_sem=send_sems_p.at[hop],
                recv_sem=recv_sems_p.at[hop],
                device_id=(right,),
                device_id_type=pl.DeviceIdType.MESH,
            )
            rdma_m = pltpu.make_async_remote_copy(
                src_ref=sbuf_m.at[hop],
                dst_ref=comm_m.at[hop],
                send_sem=send_sems_m.at[hop],
                recv_sem=recv_sems_m.at[hop],
                device_id=(left,),
                device_id_type=pl.DeviceIdType.MESH,
            )
            rdma_p.start()
            rdma_m.start()
            rdma_p.wait()
            rdma_m.wait()
            p_recv = pl.ds(
                jnp.remainder(me - hop - 1, N_DEV) * CHUNK, CHUNK
            )
            m_recv = pl.ds(
                jnp.remainder(me + hop + 1, N_DEV) * CHUNK, CHUNK
            )
            out_ref[p_recv, 0:HALF] = out_ref[p_recv, 0:HALF] + comm_p[
                hop
            ].astype(jnp.float32)
            out_ref[m_recv, HALF:D] = out_ref[m_recv, HALF:D] + comm_m[
                hop
            ].astype(jnp.float32)

        own_p = pl.ds(jnp.remainder(me + 1, N_DEV) * CHUNK, CHUNK)
        own_m = pl.ds(jnp.remainder(me - 1, N_DEV) * CHUNK, CHUNK)
        agstage_p[...] = out_ref[own_p, 0:HALF].astype(jnp.bfloat16)
        agstage_m[...] = out_ref[own_m, HALF:D].astype(jnp.bfloat16)
        for g in range(N_HOP):
            src_p = agstage_p if g == 0 else agcomm_p.at[g - 1]
            src_m = agstage_m if g == 0 else agcomm_m.at[g - 1]
            rdma_p = pltpu.make_async_remote_copy(
                src_ref=src_p,
                dst_ref=agcomm_p.at[g],
                send_sem=send_sems_p.at[N_HOP + g],
                recv_sem=recv_sems_p.at[N_HOP + g],
                device_id=(right,),
                device_id_type=pl.DeviceIdType.MESH,
            )
            rdma_m = pltpu.make_async_remote_copy(
                src_ref=src_m,
                dst_ref=agcomm_m.at[g],
                send_sem=send_sems_m.at[N_HOP + g],
                recv_sem=recv_sems_m.at[N_HOP + g],
                device_id=(left,),
                device_id_type=pl.DeviceIdType.MESH,
            )
            rdma_p.start()
            rdma_m.start()
            rdma_p.wait()
            rdma_m.wait()
            r_p = pl.ds(jnp.remainder(me - g, N_DEV) * CHUNK, CHUNK)
            r_m = pl.ds(jnp.remainder(me + g, N_DEV) * CHUNK, CHUNK)
            out_ref[r_p, 0:HALF] = agcomm_p[g].astype(jnp.float32)
            out_ref[r_m, HALF:D] = agcomm_m[g].astype(jnp.float32)

    out = pl.pallas_call(
        body,
        out_shape=jax.ShapeDtypeStruct((SQ, D), jnp.float32),
        in_specs=[pl.BlockSpec(memory_space=pltpu.VMEM)] * 8,
        out_specs=pl.BlockSpec(memory_space=pltpu.VMEM),
        scratch_shapes=[
            pltpu.VMEM((SQ, D), jnp.bfloat16),
            pltpu.VMEM((SQ, D), jnp.bfloat16),
            pltpu.VMEM((SQ, D), jnp.bfloat16),
            pltpu.VMEM((SQ, D), jnp.bfloat16),
            pltpu.VMEM((N_HOP, CHUNK, HALF), jnp.bfloat16),
            pltpu.VMEM((N_HOP, CHUNK, HALF), jnp.bfloat16),
            pltpu.VMEM((N_HOP, CHUNK, HALF), jnp.bfloat16),
            pltpu.VMEM((N_HOP, CHUNK, HALF), jnp.bfloat16),
            pltpu.VMEM((CHUNK, HALF), jnp.bfloat16),
            pltpu.VMEM((N_HOP, CHUNK, HALF), jnp.bfloat16),
            pltpu.VMEM((CHUNK, HALF), jnp.bfloat16),
            pltpu.VMEM((N_HOP, CHUNK, HALF), jnp.bfloat16),
            pltpu.SemaphoreType.DMA((2 * N_HOP,)),
            pltpu.SemaphoreType.DMA((2 * N_HOP,)),
            pltpu.SemaphoreType.DMA((2 * N_HOP,)),
            pltpu.SemaphoreType.DMA((2 * N_HOP,)),
        ],
        compiler_params=pltpu.CompilerParams(
            vmem_limit_bytes=100 * 1024 * 1024,
        ),
    )(xb, wq, wk, wv, wo, cos, sin, pmat)
    return out.reshape(1, SQ, D)


# device time: 169716 ns/iter; 1.1000x vs baseline; 1.1000x over previous
import jax
import jax.numpy as jnp
import numpy as np
from jax import lax
from jax.experimental import pallas as pl
from jax.experimental.pallas import tpu as pltpu

N_DEV = 8
SQ = 2048
D = 1024
HQ = 8
DH = 128
BLK = 1024
CHUNK = SQ // N_DEV
HALF = D // 2
SCALE = 0.08838834764831843
N_HOP = N_DEV - 1


def _rope_tables():
    inv = 1.0 / (10000.0 ** (np.arange(0, DH, 2) / DH))
    pos = np.arange(SQ)[:, None] * inv[None, :]
    cos = np.repeat(np.cos(pos), 2, axis=-1).astype(np.float32)
    sin = np.repeat(np.sin(pos), 2, axis=-1).astype(np.float32)
    P = np.zeros((DH, DH), np.float32)
    for k in range(DH // 2):
        P[2 * k + 1, 2 * k] = -1.0
        P[2 * k, 2 * k + 1] = 1.0
    return cos, sin, P


_COS, _SIN, _P = _rope_tables()


def kernel(x, Wq, Wk, Wv, Wo):
    xb = x.reshape(SQ, D).astype(jnp.bfloat16)
    wq = Wq.astype(jnp.bfloat16)
    wk = Wk.astype(jnp.bfloat16)
    wv = Wv.astype(jnp.bfloat16)
    wo = Wo.astype(jnp.bfloat16)
    cos = jnp.asarray(_COS)
    sin = jnp.asarray(_SIN)
    pmat = jnp.asarray(_P, jnp.bfloat16)

    def body(x_ref, wq_ref, wk_ref, wv_ref, wo_ref, cos_ref, sin_ref, p_ref,
             out_ref, q_ref, k_ref, v_ref, ctx_ref, sbuf_p, comm_p,
             sbuf_m, comm_m, agstage_p, agcomm_p, agstage_m, agcomm_m,
             send_sems_p, recv_sems_p, send_sems_m, recv_sems_m):
        me = lax.axis_index("i")
        right = jnp.remainder(me + 1, N_DEV)

        xv = x_ref[...]
        q_ref[...] = jnp.dot(
            xv, wq_ref[...], preferred_element_type=jnp.float32
        ).astype(jnp.bfloat16)
        k_ref[...] = jnp.dot(
            xv, wk_ref[...], preferred_element_type=jnp.float32
        ).astype(jnp.bfloat16)
        v_ref[...] = jnp.dot(
            xv, wv_ref[...], preferred_element_type=jnp.float32
        ).astype(jnp.bfloat16)

        pm = p_ref[...]
        cos_f = cos_ref[...]
        sin_f = sin_ref[...]
        for h in range(HQ):
            c0 = h * DH
            kh = k_ref[:, c0:c0 + DH]
            krot = (
                kh.astype(jnp.float32) * cos_f
                + jnp.dot(kh, pm, preferred_element_type=jnp.float32) * sin_f
            ).astype(jnp.bfloat16)
            vh = v_ref[:, c0:c0 + DH]
            for rb in range(SQ // BLK):
                r0 = rb * BLK
                qh = q_ref[r0:r0 + BLK, c0:c0 + DH]
                qrot = (
                    (
                        qh.astype(jnp.float32) * cos_f[r0:r0 + BLK, :]
                        + jnp.dot(qh, pm, preferred_element_type=jnp.float32)
                        * sin_f[r0:r0 + BLK, :]
                    )
                    * SCALE
                ).astype(jnp.bfloat16)
                s = lax.dot_general(
                    qrot, krot, (((1,), (1,)), ((), ())),
                    preferred_element_type=jnp.float32,
                )
                e = jnp.exp(s)
                denom = jnp.sum(e, axis=1, keepdims=True)
                pv = jnp.dot(
                    e.astype(jnp.bfloat16), vh,
                    preferred_element_type=jnp.float32,
                )
                ctx_ref[r0:r0 + BLK, c0:c0 + DH] = (pv / denom).astype(
                    jnp.bfloat16
                )

        out_ref[...] = jnp.dot(
            ctx_ref[...], wo_ref[...], preferred_element_type=jnp.float32
        )

        left = jnp.remainder(me - 1, N_DEV)

        for hop in range(N_HOP):
            p_send = pl.ds(jnp.remainder(me - hop, N_DEV) * CHUNK, CHUNK)
            m_send = pl.ds(jnp.remainder(me + hop, N_DEV) * CHUNK, CHUNK)
            sbuf_p[hop] = out_ref[p_send, 0:HALF].astype(jnp.bfloat16)
            sbuf_m[hop] = out_ref[m_send, HALF:D].astype(jnp.bfloat16)
            rdma_p = pltpu.make_async_remote_copy(
                src_ref=sbuf_p.at[hop],
                dst_ref=comm_p.at[hop],
                send_sem=send_sems_p.at[hop],
                recv_sem=recv_sems_p.at[hop],
                device_id=(right,),
                device_id_type=pl.DeviceIdType.MESH,
            )
            rdma_m = pltpu.make_async_remote_copy(
                src_ref=sbuf_m.at[hop],
                dst_ref=comm_m.at[hop],
                send_sem=send_sems_m.at[hop],
                recv_sem=recv_sems_m.at[hop],
                device_id=(left,),
                device_id_type=pl.DeviceIdType.MESH,
            )
            rdma_p.start()
            rdma_m.start()
            rdma_p.wait()
            rdma_m.wait()
            p_recv = pl.ds(
                jnp.remainder(me - hop - 1, N_DEV) * CHUNK, CHUNK
            )
            m_recv = pl.ds(
                jnp.remainder(me + hop + 1, N_DEV) * CHUNK, CHUNK
            )
            out_ref[p_recv, 0:HALF] = out_ref[p_recv, 0:HALF] + comm_p[
                hop
            ].astype(jnp.float32)
            out_ref[m_recv, HALF:D] = out_ref[m_recv, HALF:D] + comm_m[
                hop
            ].astype(jnp.float32)

        own_p = pl.ds(jnp.remainder(me + 1, N_DEV) * CHUNK, CHUNK)
        own_m = pl.ds(jnp.remainder(me - 1, N_DEV) * CHUNK, CHUNK)
        agstage_p[...] = out_ref[own_p, 0:HALF].astype(jnp.bfloat16)
        agstage_m[...] = out_ref[own_m, HALF:D].astype(jnp.bfloat16)
        for g in range(N_HOP):
            src_p = agstage_p if g == 0 else agcomm_p.at[g - 1]
            src_m = agstage_m if g == 0 else agcomm_m.at[g - 1]
            rdma_p = pltpu.make_async_remote_copy(
                src_ref=src_p,
                dst_ref=agcomm_p.at[g],
                send_sem=send_sems_p.at[N_HOP + g],
                recv_sem=recv_sems_p.at[N_HOP + g],
                device_id=(right,),
                device_id_type=pl.DeviceIdType.MESH,
            )
            rdma_m = pltpu.make_async_remote_copy(
                src_ref=src_m,
                dst_ref=agcomm_m.at[g],
                send_sem=send_sems_m.at[N_HOP + g],
                recv_sem=recv_sems_m.at[N_HOP + g],
                device_id=(left,),
                device_id_type=pl.DeviceIdType.MESH,
            )
            rdma_p.start()
            rdma_m.start()
            rdma_p.wait()
            rdma_m.wait()
            r_p = pl.ds(jnp.remainder(me - g, N_DEV) * CHUNK, CHUNK)
            r_m = pl.ds(jnp.remainder(me + g, N_DEV) * CHUNK, CHUNK)
            out_ref[r_p, 0:HALF] = agcomm_p[g].astype(jnp.float32)
            out_ref[r_m, HALF:D] = agcomm_m[g].astype(jnp.float32)

    out = pl.pallas_call(
        body,
        out_shape=jax.ShapeDtypeStruct((SQ, D), jnp.float32),
        in_specs=[pl.BlockSpec(memory_space=pltpu.VMEM)] * 8,
        out_specs=pl.BlockSpec(memory_space=pltpu.VMEM),
        scratch_shapes=[
            pltpu.VMEM((SQ, D), jnp.bfloat16),
            pltpu.VMEM((SQ, D), jnp.bfloat16),
            pltpu.VMEM((SQ, D), jnp.bfloat16),
            pltpu.VMEM((SQ, D), jnp.bfloat16),
            pltpu.VMEM((N_HOP, CHUNK, HALF), jnp.bfloat16),
            pltpu.VMEM((N_HOP, CHUNK, HALF), jnp.bfloat16),
            pltpu.VMEM((N_HOP, CHUNK, HALF), jnp.bfloat16),
            pltpu.VMEM((N_HOP, CHUNK, HALF), jnp.bfloat16),
            pltpu.VMEM((CHUNK, HALF), jnp.bfloat16),
            pltpu.VMEM((N_HOP, CHUNK, HALF), jnp.bfloat16),
            pltpu.VMEM((CHUNK, HALF), jnp.bfloat16),
            pltpu.VMEM((N_HOP, CHUNK, HALF), jnp.bfloat16),
            pltpu.SemaphoreType.DMA((2 * N_HOP,)),
            pltpu.SemaphoreType.DMA((2 * N_HOP,)),
            pltpu.SemaphoreType.DMA((2 * N_HOP,)),
            pltpu.SemaphoreType.DMA((2 * N_HOP,)),
        ],
        compiler_params=pltpu.CompilerParams(
            vmem_limit_bytes=100 * 1024 * 1024,
        ),
    )(xb, wq, wk, wv, wo, cos, sin, pmat)
    return out.reshape(1, SQ, D)


# device time: 148697 ns/iter; 1.2555x vs baseline; 1.1414x over previous
import jax
import jax.numpy as jnp
import numpy as np
from jax import lax
from jax.experimental import pallas as pl
from jax.experimental.pallas import tpu as pltpu

N_DEV = 8
SQ = 2048
D = 1024
HQ = 8
DH = 128
CHUNK = SQ // N_DEV
HALF = D // 2
SCALE = 0.08838834764831843
N_HOP = N_DEV - 1


def _rope_tables():
    inv = 1.0 / (10000.0 ** (np.arange(0, DH, 2) / DH))
    pos = np.arange(SQ)[:, None] * inv[None, :]
    cos = np.repeat(np.cos(pos), 2, axis=-1).astype(np.float32)
    sin = np.repeat(np.sin(pos), 2, axis=-1).astype(np.float32)
    P = np.zeros((DH, DH), np.float32)
    for k in range(DH // 2):
        P[2 * k + 1, 2 * k] = -1.0
        P[2 * k, 2 * k + 1] = 1.0
    return cos, sin, P


_COS, _SIN, _P = _rope_tables()


def kernel(x, Wq, Wk, Wv, Wo):
    xb = x.reshape(SQ, D).astype(jnp.bfloat16)
    wq = Wq.astype(jnp.bfloat16)
    wk = Wk.astype(jnp.bfloat16)
    wv = Wv.astype(jnp.bfloat16)
    wo = Wo.astype(jnp.bfloat16)
    cos = jnp.asarray(_COS)
    sin = jnp.asarray(_SIN)
    pmat = jnp.asarray(_P, jnp.bfloat16)

    def body(x_ref, wq_ref, wk_ref, wv_ref, wo_ref, cos_ref, sin_ref, p_ref,
             out_ref, k_ref, v_ref, ctx_ref, rs_sbuf, rs_comm,
             agstage_p, agcomm_p, agstage_m, agcomm_m,
             send_sems_p, recv_sems_p, send_sems_m, recv_sems_m):
        me = lax.axis_index("i")
        right = jnp.remainder(me + 1, N_DEV)
        left = jnp.remainder(me - 1, N_DEV)

        pm = p_ref[...]
        cos_f = cos_ref[...]
        sin_f = sin_ref[...]

        xv = x_ref[...]
        k_ref[...] = jnp.dot(
            xv, wk_ref[...], preferred_element_type=jnp.float32
        ).astype(jnp.bfloat16)
        v_ref[...] = jnp.dot(
            xv, wv_ref[...], preferred_element_type=jnp.float32
        ).astype(jnp.bfloat16)
        for h in range(HQ):
            c0 = h * DH
            kh = k_ref[:, c0:c0 + DH]
            k_ref[:, c0:c0 + DH] = (
                kh.astype(jnp.float32) * cos_f
                + jnp.dot(kh, pm, preferred_element_type=jnp.float32) * sin_f
            ).astype(jnp.bfloat16)

        def compute_chunk(j):
            cst = jnp.remainder(me - j, N_DEV) * CHUNK
            rows = pl.ds(cst, CHUNK)
            xc = x_ref[rows, :]
            qc = jnp.dot(xc, wq_ref[...], preferred_element_type=jnp.float32)
            cos_c = cos_ref[rows, :]
            sin_c = sin_ref[rows, :]
            for h in range(HQ):
                c0 = h * DH
                qh = qc[:, c0:c0 + DH]
                qrot = (
                    (
                        qh * cos_c
                        + jnp.dot(
                            qh.astype(jnp.bfloat16), pm,
                            preferred_element_type=jnp.float32,
                        ) * sin_c
                    ) * SCALE
                ).astype(jnp.bfloat16)
                s = lax.dot_general(
                    qrot, k_ref[:, c0:c0 + DH], (((1,), (1,)), ((), ())),
                    preferred_element_type=jnp.float32,
                )
                e = jnp.exp(s)
                denom = jnp.sum(e, axis=1, keepdims=True)
                pv = jnp.dot(
                    e.astype(jnp.bfloat16), v_ref[:, c0:c0 + DH],
                    preferred_element_type=jnp.float32,
                )
                ctx_ref[rows, c0:c0 + DH] = (pv / denom).astype(jnp.bfloat16)
            out_ref[rows, :] = jnp.dot(
                ctx_ref[rows, :], wo_ref[...],
                preferred_element_type=jnp.float32,
            )
            return rows

        def rs_rdma(hop):
            return pltpu.make_async_remote_copy(
                src_ref=rs_sbuf.at[hop],
                dst_ref=rs_comm.at[hop],
                send_sem=send_sems_p.at[hop],
                recv_sem=recv_sems_p.at[hop],
                device_id=(right,),
                device_id_type=pl.DeviceIdType.MESH,
            )

        rows0 = compute_chunk(0)
        rs_sbuf[0] = out_ref[rows0, :].astype(jnp.bfloat16)
        rdma = rs_rdma(0)
        rdma.start()
        for j in range(1, N_DEV):
            rows = compute_chunk(j)
            rdma.wait()
            out_ref[rows, :] = out_ref[rows, :] + rs_comm[j - 1].astype(
                jnp.float32
            )
            if j < N_HOP:
                rs_sbuf[j] = out_ref[rows, :].astype(jnp.bfloat16)
                rdma = rs_rdma(j)
                rdma.start()

        own = pl.ds(jnp.remainder(me + 1, N_DEV) * CHUNK, CHUNK)
        agstage_p[...] = out_ref[own, 0:HALF].astype(jnp.bfloat16)
        agstage_m[...] = out_ref[own, HALF:D].astype(jnp.bfloat16)
        for g in range(N_HOP):
            src_p = agstage_p if g == 0 else agcomm_p.at[g - 1]
            src_m = agstage_m if g == 0 else agcomm_m.at[g - 1]
            rdma_p = pltpu.make_async_remote_copy(
                src_ref=src_p,
                dst_ref=agcomm_p.at[g],
                send_sem=send_sems_p.at[N_HOP + g],
                recv_sem=recv_sems_p.at[N_HOP + g],
                device_id=(right,),
                device_id_type=pl.DeviceIdType.MESH,
            )
            rdma_m = pltpu.make_async_remote_copy(
                src_ref=src_m,
                dst_ref=agcomm_m.at[g],
                send_sem=send_sems_m.at[g],
                recv_sem=recv_sems_m.at[g],
                device_id=(left,),
                device_id_type=pl.DeviceIdType.MESH,
            )
            rdma_p.start()
            rdma_m.start()
            rdma_p.wait()
            rdma_m.wait()
            r_p = pl.ds(jnp.remainder(me - g, N_DEV) * CHUNK, CHUNK)
            r_m = pl.ds(jnp.remainder(me + 2 + g, N_DEV) * CHUNK, CHUNK)
            out_ref[r_p, 0:HALF] = agcomm_p[g].astype(jnp.float32)
            out_ref[r_m, HALF:D] = agcomm_m[g].astype(jnp.float32)

    out = pl.pallas_call(
        body,
        out_shape=jax.ShapeDtypeStruct((SQ, D), jnp.float32),
        in_specs=[pl.BlockSpec(memory_space=pltpu.VMEM)] * 8,
        out_specs=pl.BlockSpec(memory_space=pltpu.VMEM),
        scratch_shapes=[
            pltpu.VMEM((SQ, D), jnp.bfloat16),
            pltpu.VMEM((SQ, D), jnp.bfloat16),
            pltpu.VMEM((SQ, D), jnp.bfloat16),
            pltpu.VMEM((N_HOP, CHUNK, D), jnp.bfloat16),
            pltpu.VMEM((N_HOP, CHUNK, D), jnp.bfloat16),
            pltpu.VMEM((CHUNK, HALF), jnp.bfloat16),
            pltpu.VMEM((N_HOP, CHUNK, HALF), jnp.bfloat16),
            pltpu.VMEM((CHUNK, HALF), jnp.bfloat16),
            pltpu.VMEM((N_HOP, CHUNK, HALF), jnp.bfloat16),
            pltpu.SemaphoreType.DMA((2 * N_HOP,)),
            pltpu.SemaphoreType.DMA((2 * N_HOP,)),
            pltpu.SemaphoreType.DMA((N_HOP,)),
            pltpu.SemaphoreType.DMA((N_HOP,)),
        ],
        compiler_params=pltpu.CompilerParams(
            vmem_limit_bytes=100 * 1024 * 1024,
        ),
    )(xb, wq, wk, wv, wo, cos, sin, pmat)
    return out.reshape(1, SQ, D)


# device time: 144511 ns/iter; 1.2919x vs baseline; 1.0290x over previous
import jax
import jax.numpy as jnp
import numpy as np
from jax import lax
from jax.experimental import pallas as pl
from jax.experimental.pallas import tpu as pltpu

N_DEV = 8
SQ = 2048
D = 1024
HQ = 8
DH = 128
CHUNK = SQ // N_DEV
HALF = D // 2
SCALE = 0.08838834764831843
N_HOP = N_DEV - 1


def _rope_tables():
    inv = 1.0 / (10000.0 ** (np.arange(0, DH, 2) / DH))
    pos = np.arange(SQ)[:, None] * inv[None, :]
    cos = np.repeat(np.cos(pos), 2, axis=-1).astype(np.float32)
    sin = np.repeat(np.sin(pos), 2, axis=-1).astype(np.float32)
    P = np.zeros((DH, DH), np.float32)
    for k in range(DH // 2):
        P[2 * k + 1, 2 * k] = -1.0
        P[2 * k, 2 * k + 1] = 1.0
    return cos, sin, P


_COS, _SIN, _P = _rope_tables()


def kernel(x, Wq, Wk, Wv, Wo):
    xb = x.reshape(SQ, D).astype(jnp.bfloat16)
    wq = Wq.astype(jnp.bfloat16)
    wk = Wk.astype(jnp.bfloat16)
    wv = Wv.astype(jnp.bfloat16)
    wo = Wo.astype(jnp.bfloat16)
    cos = jnp.asarray(_COS, jnp.bfloat16)
    sin = jnp.asarray(_SIN, jnp.bfloat16)
    pmat = jnp.asarray(_P, jnp.bfloat16)

    def body(x_ref, wq_ref, wk_ref, wv_ref, wo_ref, cos_ref, sin_ref, p_ref,
             out_ref, q_ref, k_ref, v_ref, ctx_ref, rs_comm,
             agstage_p, agcomm_p, agstage_m, agcomm_m,
             send_sems_p, recv_sems_p, send_sems_m, recv_sems_m):
        me = lax.axis_index("i")
        right = jnp.remainder(me + 1, N_DEV)
        left = jnp.remainder(me - 1, N_DEV)

        pm = p_ref[...]
        cos_f = cos_ref[...].astype(jnp.float32)
        sin_f = sin_ref[...].astype(jnp.float32)

        for r0 in range(0, SQ, SQ // 2):
            xv = x_ref[r0:r0 + SQ // 2, :]
            q_ref[r0:r0 + SQ // 2, :] = jnp.dot(
                xv, wq_ref[...], preferred_element_type=jnp.float32
            ).astype(jnp.bfloat16)
            k_ref[r0:r0 + SQ // 2, :] = jnp.dot(
                xv, wk_ref[...], preferred_element_type=jnp.float32
            ).astype(jnp.bfloat16)
            v_ref[r0:r0 + SQ // 2, :] = jnp.dot(
                xv, wv_ref[...], preferred_element_type=jnp.float32
            ).astype(jnp.bfloat16)
        for h in range(HQ):
            c0 = h * DH
            kh = k_ref[:, c0:c0 + DH]
            k_ref[:, c0:c0 + DH] = (
                kh.astype(jnp.float32) * cos_f
                + jnp.dot(kh, pm, preferred_element_type=jnp.float32) * sin_f
            ).astype(jnp.bfloat16)
            qh = q_ref[:, c0:c0 + DH]
            q_ref[:, c0:c0 + DH] = (
                (
                    qh.astype(jnp.float32) * cos_f
                    + jnp.dot(qh, pm, preferred_element_type=jnp.float32)
                    * sin_f
                ) * SCALE
            ).astype(jnp.bfloat16)

        def compute_chunk(j):
            cst = jnp.remainder(me - j, N_DEV) * CHUNK
            rows = pl.ds(cst, CHUNK)
            for h in range(HQ):
                c0 = h * DH
                s = lax.dot_general(
                    q_ref[rows, c0:c0 + DH], k_ref[:, c0:c0 + DH],
                    (((1,), (1,)), ((), ())),
                    preferred_element_type=jnp.float32,
                )
                e = jnp.exp(s)
                denom = jnp.sum(e, axis=1, keepdims=True)
                pv = jnp.dot(
                    e.astype(jnp.bfloat16), v_ref[:, c0:c0 + DH],
                    preferred_element_type=jnp.float32,
                )
                ctx_ref[:, c0:c0 + DH] = (pv / denom).astype(jnp.bfloat16)
            out_ref[rows, :] = jnp.dot(
                ctx_ref[...], wo_ref[...],
                preferred_element_type=jnp.float32,
            ).astype(jnp.bfloat16)
            return rows

        def rs_rdma(hop, rows):
            return pltpu.make_async_remote_copy(
                src_ref=out_ref.at[rows, :],
                dst_ref=rs_comm.at[hop],
                send_sem=send_sems_p.at[hop],
                recv_sem=recv_sems_p.at[hop],
                device_id=(right,),
                device_id_type=pl.DeviceIdType.MESH,
            )

        rows0 = compute_chunk(0)
        rdma = rs_rdma(0, rows0)
        rdma.start()
        for j in range(1, N_DEV):
            rows = compute_chunk(j)
            rdma.wait()
            out_ref[rows, :] = out_ref[rows, :] + rs_comm[j - 1]
            if j < N_HOP:
                rdma = rs_rdma(j, rows)
                rdma.start()

        own = pl.ds(jnp.remainder(me + 1, N_DEV) * CHUNK, CHUNK)
        agstage_p[...] = out_ref[own, 0:HALF]
        agstage_m[...] = out_ref[own, HALF:D]
        for g in range(N_HOP):
            src_p = agstage_p if g == 0 else agcomm_p.at[g - 1]
            src_m = agstage_m if g == 0 else agcomm_m.at[g - 1]
            rdma_p = pltpu.make_async_remote_copy(
                src_ref=src_p,
                dst_ref=agcomm_p.at[g],
                send_sem=send_sems_p.at[N_HOP + g],
                recv_sem=recv_sems_p.at[N_HOP + g],
                device_id=(right,),
                device_id_type=pl.DeviceIdType.MESH,
            )
            rdma_m = pltpu.make_async_remote_copy(
                src_ref=src_m,
                dst_ref=agcomm_m.at[g],
                send_sem=send_sems_m.at[g],
                recv_sem=recv_sems_m.at[g],
                device_id=(left,),
                device_id_type=pl.DeviceIdType.MESH,
            )
            rdma_p.start()
            rdma_m.start()
            rdma_p.wait()
            rdma_m.wait()
            r_p = pl.ds(jnp.remainder(me - g, N_DEV) * CHUNK, CHUNK)
            r_m = pl.ds(jnp.remainder(me + 2 + g, N_DEV) * CHUNK, CHUNK)
            out_ref[r_p, 0:HALF] = agcomm_p[g]
            out_ref[r_m, HALF:D] = agcomm_m[g]

    out = pl.pallas_call(
        body,
        out_shape=jax.ShapeDtypeStruct((SQ, D), jnp.bfloat16),
        in_specs=[pl.BlockSpec(memory_space=pltpu.VMEM)] * 8,
        out_specs=pl.BlockSpec(memory_space=pltpu.VMEM),
        scratch_shapes=[
            pltpu.VMEM((SQ, D), jnp.bfloat16),
            pltpu.VMEM((SQ, D), jnp.bfloat16),
            pltpu.VMEM((SQ, D), jnp.bfloat16),
            pltpu.VMEM((CHUNK, D), jnp.bfloat16),
            pltpu.VMEM((N_HOP, CHUNK, D), jnp.bfloat16),
            pltpu.VMEM((CHUNK, HALF), jnp.bfloat16),
            pltpu.VMEM((N_HOP, CHUNK, HALF), jnp.bfloat16),
            pltpu.VMEM((CHUNK, HALF), jnp.bfloat16),
            pltpu.VMEM((N_HOP, CHUNK, HALF), jnp.bfloat16),
            pltpu.SemaphoreType.DMA((2 * N_HOP,)),
            pltpu.SemaphoreType.DMA((2 * N_HOP,)),
            pltpu.SemaphoreType.DMA((N_HOP,)),
            pltpu.SemaphoreType.DMA((N_HOP,)),
        ],
        compiler_params=pltpu.CompilerParams(
            vmem_limit_bytes=100 * 1024 * 1024,
        ),
    )(xb, wq, wk, wv, wo, cos, sin, pmat)
    return out.reshape(1, SQ, D)


# device time: 142530 ns/iter; 1.3099x vs baseline; 1.0139x over previous
import jax
import jax.numpy as jnp
import numpy as np
from jax import lax
from jax.experimental import pallas as pl
from jax.experimental.pallas import tpu as pltpu

N_DEV = 8
SQ = 2048
D = 1024
HQ = 8
DH = 128
CHUNK = SQ // N_DEV
HALF = D // 2
SCALE = 0.08838834764831843
N_HOP = N_DEV - 1


def _rope_tables():
    inv = 1.0 / (10000.0 ** (np.arange(0, DH, 2) / DH))
    pos = np.arange(SQ)[:, None] * inv[None, :]
    cos = np.repeat(np.cos(pos), 2, axis=-1).astype(np.float32)
    sin = np.repeat(np.sin(pos), 2, axis=-1).astype(np.float32)
    P = np.zeros((DH, DH), np.float32)
    for k in range(DH // 2):
        P[2 * k + 1, 2 * k] = -1.0
        P[2 * k, 2 * k + 1] = 1.0
    return cos, sin, P


_COS, _SIN, _P = _rope_tables()


def kernel(x, Wq, Wk, Wv, Wo):
    xb = x.reshape(SQ, D).astype(jnp.bfloat16)
    wq = Wq.astype(jnp.bfloat16)
    wk = Wk.astype(jnp.bfloat16)
    wv = Wv.astype(jnp.bfloat16)
    wo = Wo.astype(jnp.bfloat16)
    cos = jnp.asarray(_COS, jnp.bfloat16)
    sin = jnp.asarray(_SIN, jnp.bfloat16)
    pmat = jnp.asarray(_P, jnp.bfloat16)

    def body(x_ref, wq_ref, wk_ref, wv_ref, wo_ref, cos_ref, sin_ref, p_ref,
             out_ref, q_ref, k_ref, v_ref, ctx_ref, rs_comm,
             agstage_p, agcomm_p, agstage_m, agcomm_m,
             send_sems_p, recv_sems_p, send_sems_m, recv_sems_m):
        me = lax.axis_index("i")
        right = jnp.remainder(me + 1, N_DEV)
        left = jnp.remainder(me - 1, N_DEV)

        pm = p_ref[...]
        cos_f = cos_ref[...].astype(jnp.float32)
        sin_f = sin_ref[...].astype(jnp.float32)

        for r0 in range(0, SQ, SQ // 2):
            xv = x_ref[r0:r0 + SQ // 2, :]
            q_ref[r0:r0 + SQ // 2, :] = jnp.dot(
                xv, wq_ref[...], preferred_element_type=jnp.float32
            ).astype(jnp.bfloat16)
            k_ref[r0:r0 + SQ // 2, :] = jnp.dot(
                xv, wk_ref[...], preferred_element_type=jnp.float32
            ).astype(jnp.bfloat16)
            v_ref[r0:r0 + SQ // 2, :] = jnp.dot(
                xv, wv_ref[...], preferred_element_type=jnp.float32
            ).astype(jnp.bfloat16)
        for h in range(HQ):
            c0 = h * DH
            kh = k_ref[:, c0:c0 + DH]
            k_ref[:, c0:c0 + DH] = (
                kh.astype(jnp.float32) * cos_f
                + jnp.dot(kh, pm, preferred_element_type=jnp.float32) * sin_f
            ).astype(jnp.bfloat16)
            qh = q_ref[:, c0:c0 + DH]
            q_ref[:, c0:c0 + DH] = (
                (
                    qh.astype(jnp.float32) * cos_f
                    + jnp.dot(qh, pm, preferred_element_type=jnp.float32)
                    * sin_f
                ) * SCALE
            ).astype(jnp.bfloat16)

        def compute_chunk(j):
            cst = jnp.remainder(me - j, N_DEV) * CHUNK
            rows = pl.ds(cst, CHUNK)
            for h in range(HQ):
                c0 = h * DH
                s = lax.dot_general(
                    q_ref[rows, c0:c0 + DH], k_ref[:, c0:c0 + DH],
                    (((1,), (1,)), ((), ())),
                    preferred_element_type=jnp.float32,
                )
                e = jnp.exp(s)
                denom = jnp.sum(e, axis=1, keepdims=True)
                pv = jnp.dot(
                    e.astype(jnp.bfloat16), v_ref[:, c0:c0 + DH],
                    preferred_element_type=jnp.float32,
                )
                ctx_ref[:, c0:c0 + DH] = (pv / denom).astype(jnp.bfloat16)
            out_ref[rows, :] = jnp.dot(
                ctx_ref[...], wo_ref[...],
                preferred_element_type=jnp.float32,
            ).astype(jnp.bfloat16)
            return rows

        def rs_rdma(hop, rows):
            return pltpu.make_async_remote_copy(
                src_ref=out_ref.at[rows, :],
                dst_ref=rs_comm.at[hop],
                send_sem=send_sems_p.at[hop],
                recv_sem=recv_sems_p.at[hop],
                device_id=(right,),
                device_id_type=pl.DeviceIdType.MESH,
            )

        rows0 = compute_chunk(0)
        rdma = rs_rdma(0, rows0)
        rdma.start()
        for j in range(1, N_DEV):
            rows = compute_chunk(j)
            rdma.wait()
            out_ref[rows, :] = out_ref[rows, :] + rs_comm[j - 1]
            if j < N_HOP:
                rdma = rs_rdma(j, rows)
                rdma.start()

        own = pl.ds(jnp.remainder(me + 1, N_DEV) * CHUNK, CHUNK)
        agstage_p[...] = out_ref[own, 0:HALF]
        agstage_m[...] = out_ref[own, HALF:D]

        def ag_rdma(g, comm, stage, ssems, rsems, off, dev):
            return pltpu.make_async_remote_copy(
                src_ref=stage if g == 0 else comm.at[g - 1],
                dst_ref=comm.at[g],
                send_sem=ssems.at[off + g],
                recv_sem=rsems.at[off + g],
                device_id=(dev,),
                device_id_type=pl.DeviceIdType.MESH,
            )

        rdma_p = ag_rdma(0, agcomm_p, agstage_p, send_sems_p, recv_sems_p,
                         N_HOP, right)
        rdma_m = ag_rdma(0, agcomm_m, agstage_m, send_sems_m, recv_sems_m,
                         0, left)
        rdma_p.start()
        rdma_m.start()
        for g in range(N_HOP):
            rdma_p.wait()
            if g + 1 < N_HOP:
                rdma_p = ag_rdma(g + 1, agcomm_p, agstage_p, send_sems_p,
                                 recv_sems_p, N_HOP, right)
                rdma_p.start()
            rdma_m.wait()
            if g + 1 < N_HOP:
                rdma_m = ag_rdma(g + 1, agcomm_m, agstage_m, send_sems_m,
                                 recv_sems_m, 0, left)
                rdma_m.start()
            r_p = pl.ds(jnp.remainder(me - g, N_DEV) * CHUNK, CHUNK)
            r_m = pl.ds(jnp.remainder(me + 2 + g, N_DEV) * CHUNK, CHUNK)
            out_ref[r_p, 0:HALF] = agcomm_p[g]
            out_ref[r_m, HALF:D] = agcomm_m[g]

    out = pl.pallas_call(
        body,
        out_shape=jax.ShapeDtypeStruct((SQ, D), jnp.bfloat16),
        in_specs=[pl.BlockSpec(memory_space=pltpu.VMEM)] * 8,
        out_specs=pl.BlockSpec(memory_space=pltpu.VMEM),
        scratch_shapes=[
            pltpu.VMEM((SQ, D), jnp.bfloat16),
            pltpu.VMEM((SQ, D), jnp.bfloat16),
            pltpu.VMEM((SQ, D), jnp.bfloat16),
            pltpu.VMEM((CHUNK, D), jnp.bfloat16),
            pltpu.VMEM((N_HOP, CHUNK, D), jnp.bfloat16),
            pltpu.VMEM((CHUNK, HALF), jnp.bfloat16),
            pltpu.VMEM((N_HOP, CHUNK, HALF), jnp.bfloat16),
            pltpu.VMEM((CHUNK, HALF), jnp.bfloat16),
            pltpu.VMEM((N_HOP, CHUNK, HALF), jnp.bfloat16),
            pltpu.SemaphoreType.DMA((2 * N_HOP,)),
            pltpu.SemaphoreType.DMA((2 * N_HOP,)),
            pltpu.SemaphoreType.DMA((N_HOP,)),
            pltpu.SemaphoreType.DMA((N_HOP,)),
        ],
        compiler_params=pltpu.CompilerParams(
            vmem_limit_bytes=100 * 1024 * 1024,
        ),
    )(xb, wq, wk, wv, wo, cos, sin, pmat)
    return out.reshape(1, SQ, D)


# device time: 135765 ns/iter; 1.3751x vs baseline; 1.0498x over previous
import jax
import jax.numpy as jnp
import numpy as np
from jax import lax
from jax.experimental import pallas as pl
from jax.experimental.pallas import tpu as pltpu

N_DEV = 8
SQ = 2048
D = 1024
HQ = 8
DH = 128
CHUNK = SQ // N_DEV
HALF = D // 2
SCALE = 0.08838834764831843
N_HOP = N_DEV - 1


def _rope_tables():
    inv = 1.0 / (10000.0 ** (np.arange(0, DH, 2) / DH))
    pos = np.arange(SQ)[:, None] * inv[None, :]
    cos = np.repeat(np.cos(pos), 2, axis=-1).astype(np.float32)
    sin = np.repeat(np.sin(pos), 2, axis=-1).astype(np.float32)
    P = np.zeros((DH, DH), np.float32)
    for k in range(DH // 2):
        P[2 * k + 1, 2 * k] = -1.0
        P[2 * k, 2 * k + 1] = 1.0
    return cos, sin, P


_COS, _SIN, _P = _rope_tables()


def kernel(x, Wq, Wk, Wv, Wo):
    xb = x.reshape(SQ, D).astype(jnp.bfloat16)
    wq = Wq.astype(jnp.bfloat16)
    wk = Wk.astype(jnp.bfloat16)
    wv = Wv.astype(jnp.bfloat16)
    wo = Wo.astype(jnp.bfloat16)
    cos = jnp.asarray(_COS, jnp.bfloat16)
    sin = jnp.asarray(_SIN, jnp.bfloat16)
    pmat = jnp.asarray(_P, jnp.bfloat16)

    def body(x_ref, wq_ref, wk_ref, wv_ref, wo_ref, cos_ref, sin_ref, p_ref,
             out_ref, q_ref, k_ref, v_ref, ctx_ref, rs_comm,
             agstage_p, agcomm_p, agstage_m, agcomm_m,
             send_sems_p, recv_sems_p, send_sems_m, recv_sems_m):
        me = lax.axis_index("i")
        right = jnp.remainder(me + 1, N_DEV)
        left = jnp.remainder(me - 1, N_DEV)

        pm = p_ref[...]
        cos_f = cos_ref[...].astype(jnp.float32)
        sin_f = sin_ref[...].astype(jnp.float32)

        for r0 in range(0, SQ, SQ // 2):
            xv = x_ref[r0:r0 + SQ // 2, :]
            q_ref[r0:r0 + SQ // 2, :] = jnp.dot(
                xv, wq_ref[...], preferred_element_type=jnp.float32
            ).astype(jnp.bfloat16)
            k_ref[r0:r0 + SQ // 2, :] = jnp.dot(
                xv, wk_ref[...], preferred_element_type=jnp.float32
            ).astype(jnp.bfloat16)
            v_ref[r0:r0 + SQ // 2, :] = jnp.dot(
                xv, wv_ref[...], preferred_element_type=jnp.float32
            ).astype(jnp.bfloat16)
        for h in range(HQ):
            c0 = h * DH
            kh = k_ref[:, c0:c0 + DH]
            k_ref[:, c0:c0 + DH] = (
                kh.astype(jnp.float32) * cos_f
                + jnp.dot(kh, pm, preferred_element_type=jnp.float32) * sin_f
            ).astype(jnp.bfloat16)
            qh = q_ref[:, c0:c0 + DH]
            q_ref[:, c0:c0 + DH] = (
                (
                    qh.astype(jnp.float32) * cos_f
                    + jnp.dot(qh, pm, preferred_element_type=jnp.float32)
                    * sin_f
                ) * SCALE
            ).astype(jnp.bfloat16)

        def compute_chunk(j):
            cst = jnp.remainder(me - j, N_DEV) * CHUNK
            rows = pl.ds(cst, CHUNK)
            for h in range(HQ):
                c0 = h * DH
                s = lax.dot_general(
                    q_ref[rows, c0:c0 + DH], k_ref[:, c0:c0 + DH],
                    (((1,), (1,)), ((), ())),
                    preferred_element_type=jnp.float32,
                )
                e = jnp.exp(s)
                denom = jnp.sum(e, axis=1, keepdims=True)
                pv = jnp.dot(
                    e.astype(jnp.bfloat16), v_ref[:, c0:c0 + DH],
                    preferred_element_type=jnp.float32,
                )
                ctx_ref[:, c0:c0 + DH] = (pv / denom).astype(jnp.bfloat16)
            out_ref[rows, :] = jnp.dot(
                ctx_ref[...], wo_ref[...],
                preferred_element_type=jnp.float32,
            ).astype(jnp.bfloat16)
            return rows

        def rs_rdma(hop, rows):
            return pltpu.make_async_remote_copy(
                src_ref=out_ref.at[rows, :],
                dst_ref=rs_comm.at[hop],
                send_sem=send_sems_p.at[hop],
                recv_sem=recv_sems_p.at[hop],
                device_id=(right,),
                device_id_type=pl.DeviceIdType.MESH,
            )

        rows0 = compute_chunk(0)
        rdma = rs_rdma(0, rows0)
        rdma.start()
        for j in range(1, N_DEV):
            rows = compute_chunk(j)
            rdma.wait()
            out_ref[rows, :] = out_ref[rows, :] + rs_comm[j - 1]
            if j < N_HOP:
                rdma = rs_rdma(j, rows)
                rdma.start()

        own = pl.ds(jnp.remainder(me + 1, N_DEV) * CHUNK, CHUNK)
        agstage_p[...] = out_ref[own, 0:HALF]
        agstage_m[...] = out_ref[own, HALF:D]

        SUB = CHUNK // 2

        def ag_rdma(g, sub, comm, stage, ssems, rsems, off, dev):
            lo, hi = sub * SUB, (sub + 1) * SUB
            return pltpu.make_async_remote_copy(
                src_ref=(stage.at[lo:hi, :] if g == 0
                         else comm.at[g - 1, lo:hi, :]),
                dst_ref=comm.at[g, lo:hi, :],
                send_sem=ssems.at[off + 2 * g + sub],
                recv_sem=rsems.at[off + 2 * g + sub],
                device_id=(dev,),
                device_id_type=pl.DeviceIdType.MESH,
            )

        def ag_p(g, sub):
            return ag_rdma(g, sub, agcomm_p, agstage_p, send_sems_p,
                           recv_sems_p, N_HOP, right)

        def ag_m(g, sub):
            return ag_rdma(g, sub, agcomm_m, agstage_m, send_sems_m,
                           recv_sems_m, 0, left)

        flows = [ag_p(0, 0), ag_p(0, 1), ag_m(0, 0), ag_m(0, 1)]
        for f in flows:
            f.start()
        for g in range(N_HOP):
            for i, (mk, sub) in enumerate(
                ((ag_p, 0), (ag_p, 1), (ag_m, 0), (ag_m, 1))
            ):
                flows[i].wait()
                if g + 1 < N_HOP:
                    flows[i] = mk(g + 1, sub)
                    flows[i].start()
            r_p = pl.ds(jnp.remainder(me - g, N_DEV) * CHUNK, CHUNK)
            r_m = pl.ds(jnp.remainder(me + 2 + g, N_DEV) * CHUNK, CHUNK)
            out_ref[r_p, 0:HALF] = agcomm_p[g]
            out_ref[r_m, HALF:D] = agcomm_m[g]

    out = pl.pallas_call(
        body,
        out_shape=jax.ShapeDtypeStruct((SQ, D), jnp.bfloat16),
        in_specs=[pl.BlockSpec(memory_space=pltpu.VMEM)] * 8,
        out_specs=pl.BlockSpec(memory_space=pltpu.VMEM),
        scratch_shapes=[
            pltpu.VMEM((SQ, D), jnp.bfloat16),
            pltpu.VMEM((SQ, D), jnp.bfloat16),
            pltpu.VMEM((SQ, D), jnp.bfloat16),
            pltpu.VMEM((CHUNK, D), jnp.bfloat16),
            pltpu.VMEM((N_HOP, CHUNK, D), jnp.bfloat16),
            pltpu.VMEM((CHUNK, HALF), jnp.bfloat16),
            pltpu.VMEM((N_HOP, CHUNK, HALF), jnp.bfloat16),
            pltpu.VMEM((CHUNK, HALF), jnp.bfloat16),
            pltpu.VMEM((N_HOP, CHUNK, HALF), jnp.bfloat16),
            pltpu.SemaphoreType.DMA((3 * N_HOP,)),
            pltpu.SemaphoreType.DMA((3 * N_HOP,)),
            pltpu.SemaphoreType.DMA((2 * N_HOP,)),
            pltpu.SemaphoreType.DMA((2 * N_HOP,)),
        ],
        compiler_params=pltpu.CompilerParams(
            vmem_limit_bytes=100 * 1024 * 1024,
        ),
    )(xb, wq, wk, wv, wo, cos, sin, pmat)
    return out.reshape(1, SQ, D)
